# baseline (speedup 1.0000x reference)
import jax
import jax.numpy as jnp
import numpy as np
from jax import lax
from jax.experimental import pallas as pl
from jax.experimental.pallas import tpu as pltpu
from jax.experimental.pallas import tpu_sc as plsc

N_NODES = 10000
N_EDGES = 320000
D = 128

NC = 2
NS = 16
NW = NC * NS
B = 128
NCHUNK = 80
EPW = NCHUNK * B
E_PAD = NW * EPW
L = 16

SUB_STRIDE = 624
SUB_SPAN = 640

_COLPERM = np.stack([np.arange(16), np.arange(16) + 16], 1).reshape(32)
COLPERM = np.concatenate([g * 32 + _COLPERM for g in range(D // 32)])


def _sc_body(x_hbm, src_hbm, dst_hbm, vals_hbm, z_hbm, part_hbm,
             rbf0, rbf1, srows0, srows1,
             srcb0, srcb1, dstb0, dstb1, valb0, valb1,
             acc, zs, si0, si1, di0, di1, vi0, vi1, g0, g1, s0, s1):
    c = lax.axis_index("c")
    s = lax.axis_index("s")
    wid = c * NS + s
    base_row = pl.multiple_of(s * SUB_STRIDE, 8)

    pltpu.async_copy(z_hbm, acc.at[pl.ds(base_row, SUB_SPAN)], zs)

    pltpu.async_copy(src_hbm.at[wid, 0], srcb0, si0)
    pltpu.async_copy(src_hbm.at[wid, 1], srcb1, si1)
    pltpu.async_copy(dst_hbm.at[wid, 0], dstb0, di0)
    pltpu.async_copy(vals_hbm.at[wid, 0], valb0, vi0)
    pltpu.make_async_copy(src_hbm.at[wid, 0], srcb0, si0).wait()
    pltpu.async_copy(x_hbm.at[srcb0], rbf0, g0)
    with jax.named_scope("zinit"):
        pltpu.make_async_copy(z_hbm, acc.at[pl.ds(base_row, SUB_SPAN)],
                              zs).wait()
    plsc.subcore_barrier()

    bufs = ((srcb0, dstb0, valb0, rbf0, srows0, si0, di0, vi0, g0, s0),
            (srcb1, dstb1, valb1, rbf1, srows1, si1, di1, vi1, g1, s1))
    NH = NCHUNK // 2

    @pl.loop(0, NH)
    def _pair(k):
        for phase in range(2):
            j = 2 * k + phase
            srcb, dstb, valb, rbf, srows, si, di, vi, g, sc = bufs[phase]
            qsrcb, qdstb, qvalb, qrbf, qsrows, qsi, qdi, qvi, qg, qsc = (
                bufs[1 - phase])

            with jax.named_scope("g_wait"):
                pltpu.make_async_copy(x_hbm.at[srcb], rbf, g).wait()

            @pl.when(k < NH - 1)
            def _():
                pltpu.async_copy(src_hbm.at[wid, j + 2], srcb, si)

            with jax.named_scope("s_wait"):
                if phase == 0:
                    @pl.when(k >= 1)
                    def _():
                        pltpu.make_async_copy(qsrows, acc.at[qdstb],
                                              qsc).wait()
                else:
                    pltpu.make_async_copy(qsrows, acc.at[qdstb], qsc).wait()

            if phase == 0:
                pltpu.async_copy(dst_hbm.at[wid, j + 1], qdstb, qdi)
                pltpu.async_copy(vals_hbm.at[wid, j + 1], qvalb, qvi)
                pltpu.make_async_copy(src_hbm.at[wid, j + 1], qsrcb, qsi).wait()
                pltpu.async_copy(x_hbm.at[qsrcb], qrbf, qg)
            else:
                @pl.when(k < NH - 1)
                def _():
                    pltpu.async_copy(dst_hbm.at[wid, j + 1], qdstb, qdi)
                    pltpu.async_copy(vals_hbm.at[wid, j + 1], qvalb, qvi)
                    pltpu.make_async_copy(src_hbm.at[wid, j + 1],
                                          qsrcb, qsi).wait()
                    pltpu.async_copy(x_hbm.at[qsrcb], qrbf, qg)

            pltpu.make_async_copy(vals_hbm.at[wid, j], valb, vi).wait()

            with jax.named_scope("scale"):
                @plsc.parallel_loop(0, B, step=1, unroll=4)
                def _scale(i):
                    sp = plsc.load_gather(valb, [jnp.full((L,), i, jnp.int32)])
                    for g2 in range(D // 32):
                        ab = plsc.bitcast(rbf[i, pl.ds(L * g2, L)],
                                          jnp.bfloat16)
                        a, b = plsc.unpack(
                            ab, format=plsc.PackFormat.INTERLEAVED)
                        srows[i, pl.ds(32 * g2, L)] = a * sp
                        srows[i, pl.ds(32 * g2 + L, L)] = b * sp

            pltpu.make_async_copy(dst_hbm.at[wid, j], dstb, di).wait()
            pltpu.async_copy(srows, acc.at[dstb], sc, add=True)

    pltpu.make_async_copy(srows1, acc.at[dstb1], s1).wait()

    plsc.subcore_barrier()
    with jax.named_scope("writeout"):
        pltpu.sync_copy(acc.at[pl.ds(base_row, SUB_SPAN)],
                        part_hbm.at[c, pl.ds(base_row, SUB_SPAN)])


@jax.jit
def _spmm_sc(xi, src3, dst3, vals3, zblk):
    mesh = plsc.VectorSubcoreMesh(core_axis_name="c", subcore_axis_name="s",
                                  num_cores=NC, num_subcores=NS)
    return pl.kernel(
        _sc_body,
        out_type=jax.ShapeDtypeStruct((NC, N_NODES, D), jnp.float32),
        mesh=mesh,
        scratch_types=[
            pltpu.VMEM((B, D // 2), jnp.int32),
            pltpu.VMEM((B, D // 2), jnp.int32),
            pltpu.VMEM((B, D), jnp.float32),
            pltpu.VMEM((B, D), jnp.float32),
            pltpu.VMEM((B,), jnp.int32),
            pltpu.VMEM((B,), jnp.int32),
            pltpu.VMEM((B,), jnp.int32),
            pltpu.VMEM((B,), jnp.int32),
            pltpu.VMEM((B,), jnp.float32),
            pltpu.VMEM((B,), jnp.float32),
            pltpu.VMEM_SHARED((N_NODES, D), jnp.float32),
        ] + [pltpu.SemaphoreType.DMA] * 11,
        compiler_params=pltpu.CompilerParams(needs_layout_passes=False, use_tc_tiling_on_sc=False),
    )(xi, src3, dst3, vals3, zblk)


def _combine_body(p0_ref, p1_ref, e_ref, o_ref):
    o_ref[...] = p0_ref[...] + p1_ref[...] + e_ref[...]


@jax.jit
def _combine(p0, p1, e):
    grid = 10
    rows = N_NODES // grid
    spec = pl.BlockSpec((rows, D), lambda i: (i, 0))
    return pl.pallas_call(
        _combine_body,
        out_shape=jax.ShapeDtypeStruct((N_NODES, D), jnp.float32),
        grid=(grid,),
        in_specs=[spec, spec, spec],
        out_specs=spec,
    )(p0, p1, e)


def kernel(t, x, edge_index, adj_vals, e):
    src = edge_index[0].astype(jnp.int32)
    dst = edge_index[1].astype(jnp.int32)
    pad = E_PAD - N_EDGES
    spread = (jnp.arange(pad, dtype=jnp.int32) * 79) % N_NODES
    src = jnp.concatenate([src, spread])
    dst = jnp.concatenate([dst, spread])
    vals = jnp.concatenate([adj_vals, jnp.zeros((pad,), jnp.float32)])
    src3 = src.reshape(NW, NCHUNK, B)
    dst3 = dst.reshape(NW, NCHUNK, B)
    vals3 = vals.reshape(NW, NCHUNK, B)
    zblk = jnp.zeros((SUB_SPAN, D), jnp.float32)
    xb = x.astype(jnp.bfloat16)[:, COLPERM]
    xi = lax.bitcast_convert_type(xb.reshape(N_NODES, D // 2, 2), jnp.int32)
    parts = _spmm_sc(xi, src3, dst3, vals3, zblk)
    return _combine(parts[0], parts[1], e)


# speedup vs baseline: 1.0818x; 1.0818x over previous
import jax
import jax.numpy as jnp
from jax import lax
from jax.experimental import pallas as pl
from jax.experimental.pallas import tpu as pltpu
from jax.experimental.pallas import tpu_sc as plsc

N_NODES = 10000
N_EDGES = 320000
D = 128

NC = 2
NS = 16
NW = NC * NS
B = 128
NCHUNK = 80
EPW = NCHUNK * B
E_PAD = NW * EPW
L = 16

SUB_STRIDE = 624
SUB_SPAN = 640


def _sc_body(x_hbm, src_hbm, dst_hbm, vals_hbm, z_hbm, part_hbm,
             valsv, rows0, rows1, srcb0, srcb1, dstb0, dstb1,
             acc, zs, si0, si1, di0, di1, g0, g1, s0, s1):
    c = lax.axis_index("c")
    s = lax.axis_index("s")
    wid = c * NS + s
    base_row = pl.multiple_of(s * SUB_STRIDE, 8)
    xc = x_hbm.at[c]

    pltpu.async_copy(z_hbm, acc.at[pl.ds(base_row, SUB_SPAN)], zs)

    pltpu.async_copy(src_hbm.at[wid, 0], srcb0, si0)
    pltpu.async_copy(src_hbm.at[wid, 1], srcb1, si1)
    pltpu.async_copy(dst_hbm.at[wid, 0], dstb0, di0)
    pltpu.make_async_copy(src_hbm.at[wid, 0], srcb0, si0).wait()
    pltpu.async_copy(xc.at[srcb0], rows0, g0)
    pltpu.sync_copy(vals_hbm.at[wid], valsv)
    pltpu.make_async_copy(z_hbm, acc.at[pl.ds(base_row, SUB_SPAN)], zs).wait()
    plsc.subcore_barrier()

    bufs = ((srcb0, dstb0, rows0, si0, di0, g0, s0),
            (srcb1, dstb1, rows1, si1, di1, g1, s1))
    NH = NCHUNK // 2

    @pl.loop(0, NH)
    def _pair(k):
        for phase in range(2):
            j = 2 * k + phase
            srcb, dstb, rows, si, di, g, sc = bufs[phase]
            qsrcb, qdstb, qrows, qsi, qdi, qg, qsc = bufs[1 - phase]

            pltpu.make_async_copy(xc.at[srcb], rows, g).wait()

            @pl.when(k < NH - 1)
            def _():
                pltpu.async_copy(src_hbm.at[wid, j + 2], srcb, si)

            if phase == 0:
                @pl.when(k >= 1)
                def _():
                    pltpu.make_async_copy(qrows, acc.at[qdstb], qsc).wait()
            else:
                pltpu.make_async_copy(qrows, acc.at[qdstb], qsc).wait()

            if phase == 0:
                pltpu.async_copy(dst_hbm.at[wid, j + 1], qdstb, qdi)
                pltpu.make_async_copy(src_hbm.at[wid, j + 1], qsrcb, qsi).wait()
                pltpu.async_copy(xc.at[qsrcb], qrows, qg)
            else:
                @pl.when(k < NH - 1)
                def _():
                    pltpu.async_copy(dst_hbm.at[wid, j + 1], qdstb, qdi)
                    pltpu.make_async_copy(src_hbm.at[wid, j + 1],
                                          qsrcb, qsi).wait()
                    pltpu.async_copy(xc.at[qsrcb], qrows, qg)

            jvec = jnp.full((L,), j, jnp.int32)

            @plsc.parallel_loop(0, B, step=1, unroll=4)
            def _scale(i):
                sp = plsc.load_gather(
                    valsv, [jvec, jnp.full((L,), i, jnp.int32)])
                for gg in range(D // L):
                    sl = (i, pl.ds(gg * L, L))
                    rows[sl] = rows[sl] * sp

            pltpu.make_async_copy(dst_hbm.at[wid, j], dstb, di).wait()
            pltpu.async_copy(rows, acc.at[dstb], sc, add=True)

    pltpu.make_async_copy(rows1, acc.at[dstb1], s1).wait()

    plsc.subcore_barrier()
    pltpu.sync_copy(acc.at[pl.ds(base_row, SUB_SPAN)],
                    part_hbm.at[c, pl.ds(base_row, SUB_SPAN)])


@jax.jit
def _spmm_sc(x, src3, dst3, vals3, zblk):
    mesh = plsc.VectorSubcoreMesh(core_axis_name="c", subcore_axis_name="s",
                                  num_cores=NC, num_subcores=NS)
    return pl.kernel(
        _sc_body,
        out_type=jax.ShapeDtypeStruct((NC, N_NODES, D), jnp.float32),
        mesh=mesh,
        scratch_types=[
            pltpu.VMEM((NCHUNK, B), jnp.float32),
            pltpu.VMEM((B, D), jnp.float32),
            pltpu.VMEM((B, D), jnp.float32),
            pltpu.VMEM((B,), jnp.int32),
            pltpu.VMEM((B,), jnp.int32),
            pltpu.VMEM((B,), jnp.int32),
            pltpu.VMEM((B,), jnp.int32),
            pltpu.VMEM_SHARED((N_NODES, D), jnp.float32),
        ] + [pltpu.SemaphoreType.DMA] * 9,
        compiler_params=pltpu.CompilerParams(needs_layout_passes=False),
    )(x, src3, dst3, vals3, zblk)


def _combine_body(p0_ref, p1_ref, e_ref, o_ref):
    o_ref[...] = p0_ref[...] + p1_ref[...] + e_ref[...]


@jax.jit
def _combine(p0, p1, e):
    grid = 10
    rows = N_NODES // grid
    spec = pl.BlockSpec((rows, D), lambda i: (i, 0))
    return pl.pallas_call(
        _combine_body,
        out_shape=jax.ShapeDtypeStruct((N_NODES, D), jnp.float32),
        grid=(grid,),
        in_specs=[spec, spec, spec],
        out_specs=spec,
        compiler_params=pltpu.CompilerParams(
            dimension_semantics=("parallel",)),
    )(p0, p1, e)


def kernel(t, x, edge_index, adj_vals, e):
    src = edge_index[0].astype(jnp.int32)
    dst = edge_index[1].astype(jnp.int32)
    pad = E_PAD - N_EDGES
    spread = (jnp.arange(pad, dtype=jnp.int32) * 79) % N_NODES
    src = jnp.concatenate([src, spread])
    dst = jnp.concatenate([dst, spread])
    vals = jnp.concatenate([adj_vals, jnp.zeros((pad,), jnp.float32)])
    src3 = src.reshape(NW, NCHUNK, B)
    dst3 = dst.reshape(NW, NCHUNK, B)
    vals3 = vals.reshape(NW, NCHUNK, B)
    zblk = jnp.zeros((SUB_SPAN, D), jnp.float32)
    x2 = jnp.stack([x, x])
    parts = _spmm_sc(x2, src3, dst3, vals3, zblk)
    return _combine(parts[0], parts[1], e)


# speedup vs baseline: 1.1594x; 1.0717x over previous
import jax
import jax.numpy as jnp
import numpy as np
from jax import lax
from jax.experimental import pallas as pl
from jax.experimental.pallas import tpu as pltpu
from jax.experimental.pallas import tpu_sc as plsc

N_NODES = 10000
N_EDGES = 320000
D = 128

NC = 2
NS = 16
NW = NC * NS
B = 128
NCHUNK = 80
EPW = NCHUNK * B
E_PAD = NW * EPW
L = 16

SUB_STRIDE = 624
SUB_SPAN = 640

_PAD = E_PAD - N_EDGES
PAD_IDX = ((np.arange(_PAD, dtype=np.int64) * 79) % N_NODES).astype(np.int32)


def _sc_body(x_hbm, src_hbm, dst_hbm, vals_hbm, z_hbm, part_hbm,
             valsv, rows0, rows1, srcb0, srcb1, dstb0, dstb1,
             acc, zs, si0, si1, di0, di1, g0, g1, s0, s1):
    c = lax.axis_index("c")
    s = lax.axis_index("s")
    wid = c * NS + s
    base_row = pl.multiple_of(s * SUB_STRIDE, 8)

    pltpu.async_copy(z_hbm, acc.at[pl.ds(base_row, SUB_SPAN)], zs)

    pltpu.async_copy(src_hbm.at[wid, 0], srcb0, si0)
    pltpu.async_copy(src_hbm.at[wid, 1], srcb1, si1)
    pltpu.async_copy(dst_hbm.at[wid, 0], dstb0, di0)
    pltpu.make_async_copy(src_hbm.at[wid, 0], srcb0, si0).wait()
    pltpu.async_copy(x_hbm.at[srcb0], rows0, g0)
    pltpu.sync_copy(vals_hbm.at[wid], valsv)
    pltpu.make_async_copy(z_hbm, acc.at[pl.ds(base_row, SUB_SPAN)], zs).wait()
    plsc.subcore_barrier()

    bufs = ((srcb0, dstb0, rows0, si0, di0, g0, s0),
            (srcb1, dstb1, rows1, si1, di1, g1, s1))
    NH = NCHUNK // 2

    @pl.loop(0, NH)
    def _pair(k):
        for phase in range(2):
            j = 2 * k + phase
            srcb, dstb, rows, si, di, g, sc = bufs[phase]
            qsrcb, qdstb, qrows, qsi, qdi, qg, qsc = bufs[1 - phase]

            pltpu.make_async_copy(x_hbm.at[srcb], rows, g).wait()

            @pl.when(k < NH - 1)
            def _():
                pltpu.async_copy(src_hbm.at[wid, j + 2], srcb, si)

            if phase == 0:
                @pl.when(k >= 1)
                def _():
                    pltpu.make_async_copy(qrows, acc.at[qdstb], qsc).wait()
            else:
                pltpu.make_async_copy(qrows, acc.at[qdstb], qsc).wait()

            if phase == 0:
                pltpu.async_copy(dst_hbm.at[wid, j + 1], qdstb, qdi)
                pltpu.make_async_copy(src_hbm.at[wid, j + 1], qsrcb, qsi).wait()
                pltpu.async_copy(x_hbm.at[qsrcb], qrows, qg)
            else:
                @pl.when(k < NH - 1)
                def _():
                    pltpu.async_copy(dst_hbm.at[wid, j + 1], qdstb, qdi)
                    pltpu.make_async_copy(src_hbm.at[wid, j + 1],
                                          qsrcb, qsi).wait()
                    pltpu.async_copy(x_hbm.at[qsrcb], qrows, qg)

            jvec = jnp.full((L,), j, jnp.int32)

            @plsc.parallel_loop(0, B, step=1, unroll=4)
            def _scale(i):
                sp = plsc.load_gather(
                    valsv, [jvec, jnp.full((L,), i, jnp.int32)])
                for gg in range(D // L):
                    sl = (i, pl.ds(gg * L, L))
                    rows[sl] = rows[sl] * sp

            pltpu.make_async_copy(dst_hbm.at[wid, j], dstb, di).wait()
            pltpu.async_copy(rows, acc.at[dstb], sc, add=True)

    pltpu.make_async_copy(rows1, acc.at[dstb1], s1).wait()

    plsc.subcore_barrier()
    pltpu.sync_copy(acc.at[pl.ds(base_row, SUB_SPAN)],
                    part_hbm.at[c, pl.ds(base_row, SUB_SPAN)])


@jax.jit
def _spmm_sc(x, src3, dst3, vals3, zblk):
    mesh = plsc.VectorSubcoreMesh(core_axis_name="c", subcore_axis_name="s",
                                  num_cores=NC, num_subcores=NS)
    return pl.kernel(
        _sc_body,
        out_type=jax.ShapeDtypeStruct((NC, N_NODES, D), jnp.float32),
        mesh=mesh,
        scratch_types=[
            pltpu.VMEM((NCHUNK, B), jnp.float32),
            pltpu.VMEM((B, D), jnp.float32),
            pltpu.VMEM((B, D), jnp.float32),
            pltpu.VMEM((B,), jnp.int32),
            pltpu.VMEM((B,), jnp.int32),
            pltpu.VMEM((B,), jnp.int32),
            pltpu.VMEM((B,), jnp.int32),
            pltpu.VMEM_SHARED((N_NODES, D), jnp.float32),
        ] + [pltpu.SemaphoreType.DMA] * 9,
        compiler_params=pltpu.CompilerParams(needs_layout_passes=False),
    )(x, src3, dst3, vals3, zblk)


def _combine_body(p0_ref, p1_ref, e_ref, o_ref):
    o_ref[...] = p0_ref[0] + p1_ref[0] + e_ref[...]


@jax.jit
def _combine(parts, e):
    grid = 10
    rows = N_NODES // grid
    spec = pl.BlockSpec((rows, D), lambda i: (i, 0))
    pspec0 = pl.BlockSpec((1, rows, D), lambda i: (0, i, 0))
    pspec1 = pl.BlockSpec((1, rows, D), lambda i: (1, i, 0))
    return pl.pallas_call(
        _combine_body,
        out_shape=jax.ShapeDtypeStruct((N_NODES, D), jnp.float32),
        grid=(grid,),
        in_specs=[pspec0, pspec1, spec],
        out_specs=spec,
        compiler_params=pltpu.CompilerParams(
            dimension_semantics=("parallel",)),
    )(parts, parts, e)


def kernel(t, x, edge_index, adj_vals, e):
    src = edge_index[0].astype(jnp.int32)
    dst = edge_index[1].astype(jnp.int32)
    spread = jnp.asarray(PAD_IDX)
    src = jnp.concatenate([src, spread])
    dst = jnp.concatenate([dst, spread])
    vals = jnp.concatenate([adj_vals, jnp.zeros((_PAD,), jnp.float32)])
    src3 = src.reshape(NW, NCHUNK, B)
    dst3 = dst.reshape(NW, NCHUNK, B)
    vals3 = vals.reshape(NW, NCHUNK, B)
    zblk = jnp.zeros((SUB_SPAN, D), jnp.float32)
    parts = _spmm_sc(x, src3, dst3, vals3, zblk)
    return _combine(parts, e)
